# Initial kernel scaffold; baseline (speedup 1.0000x reference)
import jax
import jax.numpy as jnp
from jax.experimental import pallas as pl


def kernel(mel_pr, w_ih_f, w_hh_f, b_ih_f, b_hh_f, w_ih_b, w_hh_b, b_ih_b, b_hh_b, w_mu, b_mu, w_sq, b_sq, pos_tab):
    raise NotImplementedError("write your pallas kernel here")


# speedup vs baseline: 1.5410x; 1.5410x over previous
import jax
import jax.numpy as jnp
from jax.experimental import pallas as pl
from jax.experimental.pallas import tpu as pltpu

F_IN = 142
F_PAD = 256
T_SEG = 32
N_SEG = 4
H = 128
Z = 128
D_OUT = 256
G = 3 * H


def _round_up(x, m):
    return ((x + m - 1) // m) * m


def _gru_kernel(x_ref,
                wih_ref,
                whf_ref,
                whb_ref,
                bhn_ref,
                wtail_ref,
                bpos_ref,
                out_ref,
                gx_ref):
    T, N, F = x_ref.shape

    x2d = x_ref[...].reshape(T * N, F)
    gx = jnp.dot(x2d, wih_ref[...], preferred_element_type=jnp.float32)
    gx_ref[...] = gx.reshape(T, N, 2 * G).astype(jnp.bfloat16)

    whf = whf_ref[...]
    whb = whb_ref[...]
    bhn_f = bhn_ref[0, 2 * H:G]
    bhn_b = bhn_ref[0, G + 2 * H:]

    def body(t, carry):
        h_f, h_b = carry
        gh_f = jnp.dot(h_f.astype(jnp.bfloat16), whf,
                       preferred_element_type=jnp.float32)
        gh_b = jnp.dot(h_b.astype(jnp.bfloat16), whb,
                       preferred_element_type=jnp.float32)
        gxf = gx_ref[t, :, :G]
        gxb = gx_ref[T - 1 - t, :, G:]

        r_f = jax.nn.sigmoid(gxf[:, 0:H] + gh_f[:, 0:H])
        z_f = jax.nn.sigmoid(gxf[:, H:2 * H] + gh_f[:, H:2 * H])
        n_f = jnp.tanh(gxf[:, 2 * H:] + r_f * (gh_f[:, 2 * H:] + bhn_f))
        h_f = n_f + z_f * (h_f - n_f)

        r_b = jax.nn.sigmoid(gxb[:, 0:H] + gh_b[:, 0:H])
        z_b = jax.nn.sigmoid(gxb[:, H:2 * H] + gh_b[:, H:2 * H])
        n_b = jnp.tanh(gxb[:, 2 * H:] + r_b * (gh_b[:, 2 * H:] + bhn_b))
        h_b = n_b + z_b * (h_b - n_b)
        return h_f, h_b

    h0 = jnp.zeros((N, H), jnp.float32)
    h_f, h_b = jax.lax.fori_loop(0, T, body, (h0, h0), unroll=4)

    out_ref[...] = (jnp.dot(h_f, wtail_ref[:H], preferred_element_type=jnp.float32)
                    + jnp.dot(h_b, wtail_ref[H:], preferred_element_type=jnp.float32)
                    + bpos_ref[...])


def kernel(mel_pr, w_ih_f, w_hh_f, b_ih_f, b_hh_f, w_ih_b, w_hh_b,
           b_ih_b, b_hh_b, w_mu, b_mu, w_sq, b_sq, pos_tab):
    bs, t_total, f = mel_pr.shape
    assert t_total == N_SEG * T_SEG and f == F_IN
    n = bs * N_SEG

    tile_n = min(512, _round_up(n, 8))
    n_pad = _round_up(n, tile_n)

    x = mel_pr.reshape(n, T_SEG, F_IN).astype(jnp.bfloat16)
    ones = jnp.ones((n, T_SEG, 1), jnp.bfloat16)
    x = jnp.concatenate([x, ones], axis=-1)
    x = jnp.pad(x, ((0, n_pad - n), (0, 0), (0, F_PAD - F_IN - 1)))
    x_tnf = jnp.transpose(x, (1, 0, 2))

    bih = jnp.concatenate([b_ih_f, b_ih_b], axis=1)
    bhh = jnp.concatenate([b_hh_f, b_hh_b], axis=1)
    n_cols = jnp.concatenate([jnp.zeros((1, 2 * H)), jnp.ones((1, H)),
                              jnp.zeros((1, 2 * H)), jnp.ones((1, H))], axis=1)
    b_fold = bih + bhh * (1.0 - n_cols)
    bhn = (bhh * n_cols).astype(jnp.float32)

    wih = jnp.concatenate([w_ih_f.T, w_ih_b.T], axis=1)
    wih = jnp.concatenate([wih, b_fold], axis=0)
    wih = jnp.pad(wih, ((0, F_PAD - F_IN - 1), (0, 0))).astype(jnp.bfloat16)

    whf = w_hh_f.T.astype(jnp.bfloat16)
    whb = w_hh_b.T.astype(jnp.bfloat16)

    wtail = (w_sq @ w_mu[Z:, :]).T
    btail = b_mu[:, Z:] @ w_sq.T + b_sq
    bpos = btail + jnp.tile(pos_tab, (tile_n // N_SEG, 1))

    grid = (n_pad // tile_n,)

    flops = (2 * T_SEG * n_pad * F_PAD * 2 * G
             + 2 * T_SEG * n_pad * H * 2 * G
             + 2 * n_pad * 2 * H * D_OUT)
    transcendentals = T_SEG * n_pad * 2 * G
    bytes_accessed = 2 * (T_SEG * n_pad * F_PAD + F_PAD * 2 * G + H * 2 * G) \
        + 4 * (2 * H * D_OUT + tile_n * D_OUT + n_pad * D_OUT)

    out2d = pl.pallas_call(
        _gru_kernel,
        out_shape=jax.ShapeDtypeStruct((n_pad, D_OUT), jnp.float32),
        grid=grid,
        in_specs=[
            pl.BlockSpec((T_SEG, tile_n, F_PAD), lambda i: (0, i, 0)),
            pl.BlockSpec((F_PAD, 2 * G), lambda i: (0, 0)),
            pl.BlockSpec((H, G), lambda i: (0, 0)),
            pl.BlockSpec((H, G), lambda i: (0, 0)),
            pl.BlockSpec((1, 2 * G), lambda i: (0, 0)),
            pl.BlockSpec((2 * H, D_OUT), lambda i: (0, 0)),
            pl.BlockSpec((tile_n, D_OUT), lambda i: (0, 0)),
        ],
        out_specs=pl.BlockSpec((tile_n, D_OUT), lambda i: (i, 0)),
        scratch_shapes=[pltpu.VMEM((T_SEG, tile_n, 2 * G), jnp.bfloat16)],
        compiler_params=pltpu.CompilerParams(
            dimension_semantics=("parallel",),
            vmem_limit_bytes=60 * 1024 * 1024,
        ),
        cost_estimate=pl.CostEstimate(flops=flops,
                                      transcendentals=transcendentals,
                                      bytes_accessed=bytes_accessed),
    )(x_tnf, wih, whf, whb, bhn, wtail, bpos)

    return out2d[:n].reshape(bs, N_SEG, D_OUT)


# speedup vs baseline: 1.7811x; 1.1558x over previous
import jax
import jax.numpy as jnp
from jax.experimental import pallas as pl
from jax.experimental.pallas import tpu as pltpu

F_IN = 142
F_PAD = 256
T_SEG = 32
N_SEG = 4
H = 128
Z = 128
D_OUT = 256
G = 3 * H


def _round_up(x, m):
    return ((x + m - 1) // m) * m


def _gru_kernel(x_ref,
                wih_ref,
                bfold_ref,
                whf_ref,
                whb_ref,
                bhn_ref,
                wtail_ref,
                bpos_ref,
                out_ref,
                gx_ref):
    N, T, F = x_ref.shape

    wih = wih_ref[...]
    bfold = bfold_ref[...]
    for t in range(T):
        gx_t = jnp.dot(x_ref[:, t, :], wih,
                       preferred_element_type=jnp.float32) + bfold
        gx_ref[t] = gx_t.astype(jnp.bfloat16)

    whf = whf_ref[...]
    whb = whb_ref[...]
    bhn_f = bhn_ref[0, 2 * H:G]
    bhn_b = bhn_ref[0, G + 2 * H:]

    def body(t, carry):
        h_f, h_b = carry
        gh_f = jnp.dot(h_f.astype(jnp.bfloat16), whf,
                       preferred_element_type=jnp.float32)
        gh_b = jnp.dot(h_b.astype(jnp.bfloat16), whb,
                       preferred_element_type=jnp.float32)
        gxf = gx_ref[t, :, :G]
        gxb = gx_ref[T - 1 - t, :, G:]

        r_f = jax.nn.sigmoid(gxf[:, 0:H] + gh_f[:, 0:H])
        z_f = jax.nn.sigmoid(gxf[:, H:2 * H] + gh_f[:, H:2 * H])
        n_f = jnp.tanh(gxf[:, 2 * H:] + r_f * (gh_f[:, 2 * H:] + bhn_f))
        h_f = n_f + z_f * (h_f - n_f)

        r_b = jax.nn.sigmoid(gxb[:, 0:H] + gh_b[:, 0:H])
        z_b = jax.nn.sigmoid(gxb[:, H:2 * H] + gh_b[:, H:2 * H])
        n_b = jnp.tanh(gxb[:, 2 * H:] + r_b * (gh_b[:, 2 * H:] + bhn_b))
        h_b = n_b + z_b * (h_b - n_b)
        return h_f, h_b

    h0 = jnp.zeros((N, H), jnp.float32)
    h_f, h_b = jax.lax.fori_loop(0, T, body, (h0, h0), unroll=4)

    out_ref[...] = (jnp.dot(h_f, wtail_ref[:H], preferred_element_type=jnp.float32)
                    + jnp.dot(h_b, wtail_ref[H:], preferred_element_type=jnp.float32)
                    + bpos_ref[...])


def kernel(mel_pr, w_ih_f, w_hh_f, b_ih_f, b_hh_f, w_ih_b, w_hh_b,
           b_ih_b, b_hh_b, w_mu, b_mu, w_sq, b_sq, pos_tab):
    bs, t_total, f = mel_pr.shape
    assert t_total == N_SEG * T_SEG and f == F_IN
    n = bs * N_SEG

    tile_n = min(512, _round_up(n, 8))
    n_pad = _round_up(n, tile_n)

    x = mel_pr.reshape(n, T_SEG, F_IN)
    if n_pad != n:
        x = jnp.pad(x, ((0, n_pad - n), (0, 0), (0, 0)))

    bih = jnp.concatenate([b_ih_f, b_ih_b], axis=1)
    bhh = jnp.concatenate([b_hh_f, b_hh_b], axis=1)
    n_cols = jnp.concatenate([jnp.zeros((1, 2 * H)), jnp.ones((1, H)),
                              jnp.zeros((1, 2 * H)), jnp.ones((1, H))], axis=1)
    b_fold = (bih + bhh * (1.0 - n_cols)).astype(jnp.float32)
    bhn = (bhh * n_cols).astype(jnp.float32)

    wih = jnp.concatenate([w_ih_f.T, w_ih_b.T], axis=1)

    whf = w_hh_f.T.astype(jnp.bfloat16)
    whb = w_hh_b.T.astype(jnp.bfloat16)

    wtail = (w_sq @ w_mu[Z:, :]).T
    btail = b_mu[:, Z:] @ w_sq.T + b_sq
    bpos = btail + jnp.tile(pos_tab, (tile_n // N_SEG, 1))

    grid = (n_pad // tile_n,)

    flops = (2 * T_SEG * n_pad * F_IN * 2 * G
             + 2 * T_SEG * n_pad * H * 2 * G
             + 2 * n_pad * 2 * H * D_OUT)
    transcendentals = T_SEG * n_pad * 2 * G
    bytes_accessed = 4 * T_SEG * n_pad * F_IN \
        + 2 * (F_IN * 2 * G + H * 2 * G) \
        + 4 * (2 * H * D_OUT + tile_n * D_OUT + n_pad * D_OUT)

    out2d = pl.pallas_call(
        _gru_kernel,
        out_shape=jax.ShapeDtypeStruct((n_pad, D_OUT), jnp.float32),
        grid=grid,
        in_specs=[
            pl.BlockSpec((tile_n, T_SEG, F_IN), lambda i: (i, 0, 0)),
            pl.BlockSpec((F_IN, 2 * G), lambda i: (0, 0)),
            pl.BlockSpec((1, 2 * G), lambda i: (0, 0)),
            pl.BlockSpec((H, G), lambda i: (0, 0)),
            pl.BlockSpec((H, G), lambda i: (0, 0)),
            pl.BlockSpec((1, 2 * G), lambda i: (0, 0)),
            pl.BlockSpec((2 * H, D_OUT), lambda i: (0, 0)),
            pl.BlockSpec((tile_n, D_OUT), lambda i: (0, 0)),
        ],
        out_specs=pl.BlockSpec((tile_n, D_OUT), lambda i: (i, 0)),
        scratch_shapes=[pltpu.VMEM((T_SEG, tile_n, 2 * G), jnp.bfloat16)],
        compiler_params=pltpu.CompilerParams(
            dimension_semantics=("parallel",),
            vmem_limit_bytes=63 * 1024 * 1024,
        ),
        cost_estimate=pl.CostEstimate(flops=flops,
                                      transcendentals=transcendentals,
                                      bytes_accessed=bytes_accessed),
    )(x, wih, b_fold, whf, whb, bhn, wtail, bpos)

    return out2d[:n].reshape(bs, N_SEG, D_OUT)
